# Initial kernel scaffold; baseline (speedup 1.0000x reference)
import jax
import jax.numpy as jnp
from jax.experimental import pallas as pl


def kernel(class_logit, box_regression, proposal):
    raise NotImplementedError("write your pallas kernel here")


# speedup vs baseline: 4.2035x; 4.2035x over previous
import functools
import math

import jax
import jax.numpy as jnp
from jax import lax
from jax.experimental import pallas as pl

N = 20000
NPAD = 20480
NUM_CLASSES = 21
IMG_H, IMG_W = 800.0, 1216.0
SCORE_THRESH = 0.05
NMS_THRESH = 0.5
NUM_DETECTIONS = 100
MIN_SIZE = 1.0
K_PRE = 1000
KP = 1024
DETP = 128
BBOX_XFORM_CLIP = math.log(1000.0 / 16.0)
_HI = jax.lax.Precision.HIGHEST


def _score_decode_body(logit_ref, reg_ref, prop_ref, s_ref, box_ref):
    lg = logit_ref[...]
    m = jnp.max(lg, axis=0, keepdims=True)
    e = jnp.exp(lg - m)
    z = jnp.sum(e, axis=0, keepdims=True)
    prob = e / z

    pr = prop_ref[...]
    px1, py1 = pr[0:1], pr[1:2]
    px2, py2 = pr[2:3], pr[3:4]
    w = px2 - px1
    h = py2 - py1
    cx = px1 + 0.5 * w
    cy = py1 + 0.5 * h

    for c in range(NUM_CLASSES - 1):
        d = reg_ref[c + 1]
        dx = d[0:1] / 10.0
        dy = d[1:2] / 10.0
        dw = jnp.minimum(d[2:3] / 5.0, BBOX_XFORM_CLIP)
        dh = jnp.minimum(d[3:4] / 5.0, BBOX_XFORM_CLIP)
        pcx = dx * w + cx
        pcy = dy * h + cy
        pw = jnp.exp(dw) * w
        ph = jnp.exp(dh) * h
        x1 = jnp.clip(pcx - 0.5 * pw, 0.0, IMG_W)
        y1 = jnp.clip(pcy - 0.5 * ph, 0.0, IMG_H)
        x2 = jnp.clip(pcx + 0.5 * pw, 0.0, IMG_W)
        y2 = jnp.clip(pcy + 0.5 * ph, 0.0, IMG_H)
        bw = x2 - x1
        bh = y2 - y1
        sc = prob[c + 1:c + 2]
        valid = (sc >= SCORE_THRESH) & (bw >= MIN_SIZE) & (bh >= MIN_SIZE)
        s_ref[c, :] = jnp.where(valid, sc, -1.0)[0]
        box_ref[c] = jnp.concatenate([x1, y1, x2, y2], axis=0)


def _nms_body(boxc_ref, boxr_ref, s_ref, dbox_ref, ds_ref):
    bc = boxc_ref[0]
    br = boxr_ref[0]
    s_row = s_ref[0]

    x1c, y1c = bc[:, 0:1], bc[:, 1:2]
    x2c, y2c = bc[:, 2:3], bc[:, 3:4]
    x1r, y1r = br[0:1, :], br[1:2, :]
    x2r, y2r = br[2:3, :], br[3:4, :]

    area_c = (x2c - x1c) * (y2c - y1c)
    area_r = (x2r - x1r) * (y2r - y1r)
    wx = jnp.maximum(jnp.minimum(x2c, x2r) - jnp.maximum(x1c, x1r), 0.0)
    wy = jnp.maximum(jnp.minimum(y2c, y2r) - jnp.maximum(y1c, y1r), 0.0)
    inter = wx * wy
    union = area_c + area_r - inter
    iou = inter / jnp.maximum(union, 1e-9)

    im = lax.broadcasted_iota(jnp.int32, (KP, KP), 0)
    jm = lax.broadcasted_iota(jnp.int32, (KP, KP), 1)
    upper = im < jm
    sup_f = jnp.where((iou > NMS_THRESH) & upper, 1.0, 0.0)

    def cond(c):
        return c[1]

    def body(c):
        k = c[0]
        sup = lax.dot_general(k, sup_f, (((1,), (0,)), ((), ())),
                              precision=_HI)
        kn = jnp.where(sup > 0.5, 0.0, 1.0)
        return kn, jnp.any(kn != k)

    keep0 = jnp.ones((1, KP), jnp.float32)
    keep, _ = lax.while_loop(cond, body, (keep0, True))

    kv = (keep > 0.5) & (s_row > 0.0)
    s2 = jnp.where(kv, s_row, -1.0)

    eye = jnp.where(im == jm, 1.0, 0.0)
    s2_col = lax.dot_general(eye, s2, (((1,), (1,)), ((), ())),
                             precision=_HI)
    a_mat = jnp.where((s2_col > s2) | ((s2_col == s2) & upper), 1.0, 0.0)
    ones_row = jnp.ones((1, KP), jnp.float32)
    rank = lax.dot_general(ones_row, a_mat, (((1,), (0,)), ((), ())),
                           precision=_HI)
    ri = rank.astype(jnp.int32)
    p_col = lax.broadcasted_iota(jnp.int32, (DETP, KP), 0)
    perm = jnp.where(ri == p_col, 1.0, 0.0)

    det_box = lax.dot_general(perm, bc, (((1,), (0,)), ((), ())),
                              precision=_HI)
    det_s = lax.dot_general(perm, s2_col, (((1,), (0,)), ((), ())),
                            precision=_HI)
    dvalid = det_s > 0.0
    dbox_ref[0] = jnp.where(dvalid, det_box, 0.0)
    ds_ref[0] = jnp.where(dvalid, det_s, 0.0)


@jax.jit
def kernel(class_logit, box_regression, proposal):
    logit_t = jnp.pad(class_logit.T, ((0, 0), (0, NPAD - N)))
    reg_t = jnp.pad(
        jnp.transpose(box_regression.reshape(N, NUM_CLASSES, 4), (1, 2, 0)),
        ((0, 0), (0, 0), (0, NPAD - N)))
    prop_t = jnp.pad(proposal.T, ((0, 0), (0, NPAD - N)))

    blk = 2048
    grid_a = NPAD // blk
    s_all, box_all = pl.pallas_call(
        _score_decode_body,
        grid=(grid_a,),
        in_specs=[
            pl.BlockSpec((NUM_CLASSES, blk), lambda i: (0, i)),
            pl.BlockSpec((NUM_CLASSES, 4, blk), lambda i: (0, 0, i)),
            pl.BlockSpec((4, blk), lambda i: (0, i)),
        ],
        out_specs=[
            pl.BlockSpec((NUM_CLASSES - 1, blk), lambda i: (0, i)),
            pl.BlockSpec((NUM_CLASSES - 1, 4, blk), lambda i: (0, 0, i)),
        ],
        out_shape=[
            jax.ShapeDtypeStruct((NUM_CLASSES - 1, NPAD), jnp.float32),
            jax.ShapeDtypeStruct((NUM_CLASSES - 1, 4, NPAD), jnp.float32),
        ],
    )(logit_t, reg_t, prop_t)

    top_s, top_i = jax.vmap(functools.partial(lax.top_k, k=K_PRE))(s_all)
    tb = jnp.take_along_axis(box_all, top_i[:, None, :], axis=2)
    tb = jnp.pad(tb, ((0, 0), (0, 0), (0, KP - K_PRE)))
    ts = jnp.pad(top_s, ((0, 0), (0, KP - K_PRE)), constant_values=-1.0)
    tbt = jnp.swapaxes(tb, 1, 2)
    ts3 = ts[:, None, :]

    det_box, det_s = pl.pallas_call(
        _nms_body,
        grid=(NUM_CLASSES - 1,),
        in_specs=[
            pl.BlockSpec((1, KP, 4), lambda c: (c, 0, 0)),
            pl.BlockSpec((1, 4, KP), lambda c: (c, 0, 0)),
            pl.BlockSpec((1, 1, KP), lambda c: (c, 0, 0)),
        ],
        out_specs=[
            pl.BlockSpec((1, DETP, 4), lambda c: (c, 0, 0)),
            pl.BlockSpec((1, DETP, 1), lambda c: (c, 0, 0)),
        ],
        out_shape=[
            jax.ShapeDtypeStruct((NUM_CLASSES - 1, DETP, 4), jnp.float32),
            jax.ShapeDtypeStruct((NUM_CLASSES - 1, DETP, 1), jnp.float32),
        ],
    )(tbt, tb, ts3)

    db = det_box[:, :NUM_DETECTIONS, :]
    ds = det_s[:, :NUM_DETECTIONS, 0]
    labels = jnp.broadcast_to(
        jnp.arange(1, NUM_CLASSES, dtype=jnp.float32)[:, None],
        (NUM_CLASSES - 1, NUM_DETECTIONS))
    labels = jnp.where(ds > 0.0, labels, 0.0)
    det = jnp.concatenate(
        [db.reshape(-1, 4), ds.reshape(-1, 1), labels.reshape(-1, 1)], axis=1)
    return det
